# Initial kernel scaffold; baseline (speedup 1.0000x reference)
import functools

import jax
import jax.numpy as jnp
from jax import lax
from jax.experimental import pallas as pl
from jax.experimental.pallas import tpu as pltpu
from jax.experimental.pallas import tpu_sc as plsc

B = 16384
D = 200
K = 64
L = 16

_info = plsc.get_sparse_core_info()
_NC, _NS = _info.num_cores, _info.num_subcores
_NW = _NC * _NS
_BPW = B // _NW
_LN2 = 0.6931471805599453


def _log_vec(s):
    bits = lax.bitcast_convert_type(s, jnp.int32)
    y = (bits.astype(jnp.float32) * (1.0 / (1 << 23)) - 127.0) * _LN2
    for _ in range(3):
        y = y - 1.0 + s * jnp.exp(-y)
    return y


def _body(inputs_hbm, nd_hbm, probs_hbm, out_hbm, tbl_v, nd_v, p_v, lsm_v,
          out_v):
    wid = lax.axis_index("s") * _NC + lax.axis_index("c")
    base = wid * _BPW

    pltpu.sync_copy(inputs_hbm.at[pl.ds(0, D)], tbl_v)
    pltpu.sync_copy(nd_hbm.at[pl.ds(base, _BPW)], nd_v)
    pltpu.sync_copy(probs_hbm, p_v)

    vs = [p_v[pl.ds(k * L, L)] for k in range(K // L)]
    mx = vs[0]
    for v in vs[1:]:
        mx = jnp.maximum(mx, v)
    m = jnp.broadcast_to(jnp.max(mx), (L,))
    se = jnp.exp(vs[0] - m)
    for v in vs[1:]:
        se = se + jnp.exp(v - m)
    s = jnp.broadcast_to(jnp.sum(se), (L,))
    lse = m + _log_vec(s)
    for k in range(K // L):
        lsm_v[pl.ds(k * L, L)] = vs[k] - lse

    iota = lax.iota(jnp.int32, L)
    zeros = jnp.zeros((L,), jnp.int32)
    ones = jnp.full((L,), 1, jnp.int32)
    for j in range(_BPW // L):
        rows = iota + (j * L)
        r = plsc.load_gather(nd_v, [rows, zeros])
        c = plsc.load_gather(nd_v, [rows, ones])
        x = plsc.load_gather(tbl_v, [r, c])
        out_v[pl.ds(j * L, L)] = plsc.load_gather(lsm_v, [x])

    pltpu.sync_copy(out_v, out_hbm.at[pl.ds(base, _BPW)])


@jax.jit
def _run(inputs, nd_idxs, probs):
    mesh = plsc.VectorSubcoreMesh(core_axis_name="c", subcore_axis_name="s")
    k = functools.partial(
        pl.kernel,
        mesh=mesh,
        out_type=jax.ShapeDtypeStruct((B,), jnp.float32),
        scratch_types=[
            pltpu.VMEM((D, D), jnp.int32),
            pltpu.VMEM((_BPW, 2), jnp.int32),
            pltpu.VMEM((K,), jnp.float32),
            pltpu.VMEM((K,), jnp.float32),
            pltpu.VMEM((_BPW,), jnp.float32),
        ],
    )(_body)
    return k(inputs, nd_idxs, probs)


def kernel(inputs, nd_idxs, probs):
    return _run(inputs, nd_idxs, probs).reshape(-1, 1)


# speedup vs baseline: 1.6832x; 1.6832x over previous
import functools

import jax
import jax.numpy as jnp
from jax import lax
from jax.experimental import pallas as pl
from jax.experimental.pallas import tpu as pltpu
from jax.experimental.pallas import tpu_sc as plsc

B = 16384
D = 200
K = 64
L = 16

_info = plsc.get_sparse_core_info()
_NC, _NS = _info.num_cores, _info.num_subcores
_NW = _NC * _NS
_BPW = B // _NW
_LN2 = 0.6931471805599453


def _log_vec(s):
    bits = lax.bitcast_convert_type(s, jnp.int32)
    y = (bits.astype(jnp.float32) * (1.0 / (1 << 23)) - 127.0) * _LN2
    for _ in range(3):
        y = y - 1.0 + s * jnp.exp(-y)
    return y


def _body(inputs_hbm, nd_hbm, probs_hbm, out_hbm, tbl_v, nd_v, p_v, lsm_v,
          out_v):
    wid = lax.axis_index("s") * _NC + lax.axis_index("c")
    base = wid * _BPW
    iota = lax.iota(jnp.int32, L)

    pltpu.sync_copy(inputs_hbm.at[pl.ds(0, D)], tbl_v)
    pltpu.sync_copy(nd_hbm.at[pl.ds(base, _BPW)], nd_v)
    pltpu.sync_copy(probs_hbm, p_v)

    def _xlane(vec, op):
        for sh in (8, 4, 2, 1):
            perm = vec.at[jnp.bitwise_xor(iota, sh)].get(
                mode="promise_in_bounds")
            vec = op(vec, perm)
        return vec

    vs = [p_v[pl.ds(k * L, L)] for k in range(K // L)]
    mx = vs[0]
    for v in vs[1:]:
        mx = jnp.maximum(mx, v)
    m = _xlane(mx, jnp.maximum)
    se = jnp.exp(vs[0] - m)
    for v in vs[1:]:
        se = se + jnp.exp(v - m)
    s = _xlane(se, jnp.add)
    lse = m + _log_vec(s)
    for k in range(K // L):
        lsm_v[pl.ds(k * L, L)] = vs[k] - lse

    zeros = jnp.zeros((L,), jnp.int32)
    ones = jnp.full((L,), 1, jnp.int32)
    for j in range(_BPW // L):
        rows = iota + (j * L)
        r = plsc.load_gather(nd_v, [rows, zeros])
        c = plsc.load_gather(nd_v, [rows, ones])
        x = plsc.load_gather(tbl_v, [r, c])
        out_v[pl.ds(j * L, L)] = plsc.load_gather(lsm_v, [x])

    pltpu.sync_copy(out_v, out_hbm.at[pl.ds(base, _BPW)])


@jax.jit
def _run(inputs, nd_idxs, probs):
    mesh = plsc.VectorSubcoreMesh(core_axis_name="c", subcore_axis_name="s")
    k = functools.partial(
        pl.kernel,
        mesh=mesh,
        compiler_params=pltpu.CompilerParams(needs_layout_passes=False),
        out_type=jax.ShapeDtypeStruct((B,), jnp.float32),
        scratch_types=[
            pltpu.VMEM((D, D), jnp.int32),
            pltpu.VMEM((_BPW, 2), jnp.int32),
            pltpu.VMEM((K,), jnp.float32),
            pltpu.VMEM((K,), jnp.float32),
            pltpu.VMEM((_BPW,), jnp.float32),
        ],
    )(_body)
    return k(inputs, nd_idxs, probs)


def kernel(inputs, nd_idxs, probs):
    return _run(inputs, nd_idxs, probs).reshape(-1, 1)


# speedup vs baseline: 2.5150x; 1.4941x over previous
import functools

import jax
import jax.numpy as jnp
from jax import lax
from jax.experimental import pallas as pl
from jax.experimental.pallas import tpu as pltpu
from jax.experimental.pallas import tpu_sc as plsc

B = 16384
D = 200
K = 64
L = 16
_CH = 128

_info = plsc.get_sparse_core_info()
_NC, _NS = _info.num_cores, _info.num_subcores
_NW = _NC * _NS
_BPW = B // _NW
_LN2 = 0.6931471805599453


def _log_vec(s):
    bits = lax.bitcast_convert_type(s, jnp.int32)
    y = (bits.astype(jnp.float32) * (1.0 / (1 << 23)) - 127.0) * _LN2
    for _ in range(3):
        y = y - 1.0 + s * jnp.exp(-y)
    return y


def _body(tbl_hbm, nd_hbm, probs_hbm, out_hbm, nd_v, fidx_v, gat_v, p_v,
          lsm_v, out_v, sem):
    wid = lax.axis_index("s") * _NC + lax.axis_index("c")
    iota = lax.iota(jnp.int32, L)

    pltpu.sync_copy(nd_hbm.at[pl.ds(wid * (2 * _BPW), 2 * _BPW)], nd_v)

    for j in range(_BPW // L):
        pos = (iota + (j * L)) * 2
        r = plsc.load_gather(nd_v, [pos])
        c = plsc.load_gather(nd_v, [pos + 1])
        fidx_v[pl.ds(j * L, L)] = r * D + c

    copies = [
        pltpu.async_copy(
            tbl_hbm.at[fidx_v.at[pl.ds(j * _CH, _CH)]],
            gat_v.at[pl.ds(j * _CH, _CH)],
            sem,
        )
        for j in range(_BPW // _CH)
    ]

    pltpu.sync_copy(probs_hbm, p_v)

    def _xlane(vec, op):
        for sh in (8, 4, 2, 1):
            perm = vec.at[jnp.bitwise_xor(iota, sh)].get(
                mode="promise_in_bounds")
            vec = op(vec, perm)
        return vec

    vs = [p_v[pl.ds(k * L, L)] for k in range(K // L)]
    mx = vs[0]
    for v in vs[1:]:
        mx = jnp.maximum(mx, v)
    m = _xlane(mx, jnp.maximum)
    se = jnp.exp(vs[0] - m)
    for v in vs[1:]:
        se = se + jnp.exp(v - m)
    s = _xlane(se, jnp.add)
    lse = m + _log_vec(s)
    for k in range(K // L):
        lsm_v[pl.ds(k * L, L)] = vs[k] - lse

    for cp in copies:
        cp.wait()

    for j in range(_BPW // L):
        x = gat_v[pl.ds(j * L, L)]
        out_v[pl.ds(j * L, L)] = plsc.load_gather(lsm_v, [x])

    pltpu.sync_copy(out_v, out_hbm.at[pl.ds(wid * _BPW, _BPW)])


@jax.jit
def _run(tbl, ndf, probs):
    mesh = plsc.VectorSubcoreMesh(core_axis_name="c", subcore_axis_name="s")
    k = functools.partial(
        pl.kernel,
        mesh=mesh,
        compiler_params=pltpu.CompilerParams(needs_layout_passes=False),
        out_type=jax.ShapeDtypeStruct((B,), jnp.float32),
        scratch_types=[
            pltpu.VMEM((2 * _BPW,), jnp.int32),
            pltpu.VMEM((_BPW,), jnp.int32),
            pltpu.VMEM((_BPW,), jnp.int32),
            pltpu.VMEM((K,), jnp.float32),
            pltpu.VMEM((K,), jnp.float32),
            pltpu.VMEM((_BPW,), jnp.float32),
            pltpu.SemaphoreType.DMA,
        ],
    )(_body)
    return k(tbl, ndf, probs)


def kernel(inputs, nd_idxs, probs):
    tbl = inputs[:D].reshape(-1)
    ndf = nd_idxs.reshape(-1)
    return _run(tbl, ndf, probs).reshape(-1, 1)


# speedup vs baseline: 3.5710x; 1.4199x over previous
import functools

import jax
import jax.numpy as jnp
from jax import lax
from jax.experimental import pallas as pl
from jax.experimental.pallas import tpu as pltpu
from jax.experimental.pallas import tpu_sc as plsc

B = 16384
D = 200
K = 64
L = 16
_CH = 128

_info = plsc.get_sparse_core_info()
_NC, _NS = _info.num_cores, _info.num_subcores
_NW = _NC * _NS
_BPW = B // _NW
_LN2 = 0.6931471805599453


def _log_vec(s):
    bits = lax.bitcast_convert_type(s, jnp.int32)
    y = (bits.astype(jnp.float32) * (1.0 / (1 << 23)) - 127.0) * _LN2
    for _ in range(3):
        y = y - 1.0 + s * jnp.exp(-y)
    return y


def _body(tbl_hbm, r_hbm, c_hbm, probs_hbm, out_hbm, r_v, c_v, fidx_v, gat_v,
          p_v, lsm_v, out_v, sem):
    wid = lax.axis_index("s") * _NC + lax.axis_index("c")
    iota = lax.iota(jnp.int32, L)

    pltpu.sync_copy(r_hbm.at[pl.ds(wid * _BPW, _BPW)], r_v)
    pltpu.sync_copy(c_hbm.at[pl.ds(wid * _BPW, _BPW)], c_v)

    for j in range(_BPW // L):
        sl = pl.ds(j * L, L)
        fidx_v[sl] = r_v[sl] * D + c_v[sl]

    copies = [
        pltpu.async_copy(
            tbl_hbm.at[fidx_v.at[pl.ds(j * _CH, _CH)]],
            gat_v.at[pl.ds(j * _CH, _CH)],
            sem,
        )
        for j in range(_BPW // _CH)
    ]

    pltpu.sync_copy(probs_hbm, p_v)

    def _xlane(vec, op):
        for sh in (8, 4, 2, 1):
            perm = vec.at[jnp.bitwise_xor(iota, sh)].get(
                mode="promise_in_bounds")
            vec = op(vec, perm)
        return vec

    vs = [p_v[pl.ds(k * L, L)] for k in range(K // L)]
    mx = vs[0]
    for v in vs[1:]:
        mx = jnp.maximum(mx, v)
    m = _xlane(mx, jnp.maximum)
    se = jnp.exp(vs[0] - m)
    for v in vs[1:]:
        se = se + jnp.exp(v - m)
    s = _xlane(se, jnp.add)
    lse = m + _log_vec(s)
    for k in range(K // L):
        lsm_v[pl.ds(k * L, L)] = vs[k] - lse

    for cp in copies:
        cp.wait()

    for j in range(_BPW // L):
        x = gat_v[pl.ds(j * L, L)]
        out_v[pl.ds(j * L, L)] = plsc.load_gather(lsm_v, [x])

    pltpu.sync_copy(out_v, out_hbm.at[pl.ds(wid * _BPW, _BPW)])


@jax.jit
def _run(tbl, r, c, probs):
    mesh = plsc.VectorSubcoreMesh(core_axis_name="c", subcore_axis_name="s")
    k = functools.partial(
        pl.kernel,
        mesh=mesh,
        compiler_params=pltpu.CompilerParams(needs_layout_passes=False),
        out_type=jax.ShapeDtypeStruct((B,), jnp.float32),
        scratch_types=[
            pltpu.VMEM((_BPW,), jnp.int32),
            pltpu.VMEM((_BPW,), jnp.int32),
            pltpu.VMEM((_BPW,), jnp.int32),
            pltpu.VMEM((_BPW,), jnp.int32),
            pltpu.VMEM((K,), jnp.float32),
            pltpu.VMEM((K,), jnp.float32),
            pltpu.VMEM((_BPW,), jnp.float32),
            pltpu.SemaphoreType.DMA,
        ],
    )(_body)
    return k(tbl, r, c, probs)


def kernel(inputs, nd_idxs, probs):
    tbl = inputs[:D].reshape(-1)
    return _run(tbl, nd_idxs[:, 0], nd_idxs[:, 1], probs).reshape(-1, 1)
